# baseline (speedup 1.0000x reference)
import jax
import jax.numpy as jnp
from jax.experimental import pallas as pl
from jax.experimental.pallas import tpu as pltpu

_NW = 4
_XC = 4096 // _NW
_WC = 512 // _NW
_BM = 256
_NB = 4096 // _BM
_BM2 = 1024
_NB2 = 4096 // _BM2


def _body(adj_ref, x_ref, w1_ref, b1_ref, w2_ref, b2_ref, out_ref,
          adjbf_ref, g_ref, xbf_ref, w1bf_ref, w2bf_ref):
    i = pl.program_id(0)

    @pl.when(i == 0)
    def _cast_w2():
        w2bf_ref[...] = w2_ref[...].astype(jnp.bfloat16)

    @pl.when(i < _NW)
    def _warmup():
        xbf_ref[pl.ds(i * _XC, _XC), :] = x_ref[...].astype(jnp.bfloat16)
        w1bf_ref[pl.ds(i * _WC, _WC), :] = w1_ref[...].astype(jnp.bfloat16)

    @pl.when(jnp.logical_and(i >= _NW, i < _NW + _NB))
    def _phase_a():
        k = i - _NW
        adjbf_ref[pl.ds(k * _BM, _BM), :] = adj_ref[...].astype(jnp.bfloat16)
        t = jnp.dot(adjbf_ref[pl.ds(k * _BM, _BM), :], xbf_ref[...],
                    preferred_element_type=jnp.float32)
        u = jnp.dot(t.astype(jnp.bfloat16), w1bf_ref[...],
                    preferred_element_type=jnp.float32) + b1_ref[...]
        h = jnp.maximum(u, 0.0)
        g = jnp.dot(h.astype(jnp.bfloat16), w2bf_ref[...],
                    preferred_element_type=jnp.float32)
        g_ref[pl.ds(k * _BM, _BM), :] = g.astype(jnp.bfloat16)

    @pl.when(i >= _NW + _NB)
    def _phase_b():
        k = i - _NW - _NB
        a = adjbf_ref[pl.ds(k * _BM2, _BM2), :]
        acc = jnp.dot(a, g_ref[...], preferred_element_type=jnp.float32)
        out_ref[...] = jnp.tanh(acc + b2_ref[...])


def kernel(x, adj, W1, b1, W2, b2):
    n, d_in = x.shape
    d_hid = W1.shape[1]
    bit = W2.shape[1]
    b1r = b1.reshape(1, d_hid)
    b2r = b2.reshape(1, bit)

    out = pl.pallas_call(
        _body,
        grid=(_NW + _NB + _NB2,),
        in_specs=[
            pl.BlockSpec((_BM, n), lambda i: (jnp.clip(i - _NW, 0, _NB - 1), 0)),
            pl.BlockSpec((_XC, d_in), lambda i: (jnp.minimum(i, _NW - 1), 0)),
            pl.BlockSpec((_WC, d_hid), lambda i: (jnp.minimum(i, _NW - 1), 0)),
            pl.BlockSpec((1, d_hid), lambda i: (0, 0)),
            pl.BlockSpec((d_hid, bit), lambda i: (0, 0)),
            pl.BlockSpec((1, bit), lambda i: (0, 0)),
        ],
        out_specs=pl.BlockSpec((_BM2, bit),
                               lambda i: (jnp.clip(i - _NW - _NB, 0, _NB2 - 1), 0)),
        out_shape=jax.ShapeDtypeStruct((n, bit), jnp.float32),
        compiler_params=pltpu.CompilerParams(vmem_limit_bytes=100 * 1024 * 1024),
        scratch_shapes=[
            pltpu.VMEM((n, n), jnp.bfloat16),
            pltpu.VMEM((n, bit), jnp.bfloat16),
            pltpu.VMEM((n, d_in), jnp.bfloat16),
            pltpu.VMEM((d_in, d_hid), jnp.bfloat16),
            pltpu.VMEM((d_hid, bit), jnp.bfloat16),
        ],
    )(adj, x, W1, b1r, W2, b2r)
    return out


# speedup vs baseline: 1.0165x; 1.0165x over previous
import jax
import jax.numpy as jnp
from jax.experimental import pallas as pl
from jax.experimental.pallas import tpu as pltpu

_NW = 4
_XC = 4096 // _NW
_WC = 512 // _NW
_BM = 256
_NB = 4096 // _BM
_BM2 = 1024
_NB2 = 4096 // _BM2


def _body(adj_ref, x_ref, w1_ref, b1_ref, w2_ref, b2_ref, out_ref,
          adjbf_ref, g_ref, xbf_ref, w1bf_ref):
    i = pl.program_id(0)

    @pl.when(i < _NW)
    def _warmup():
        xbf_ref[pl.ds(i * _XC, _XC), :] = x_ref[...].astype(jnp.bfloat16)
        w1bf_ref[pl.ds(i * _WC, _WC), :] = w1_ref[...].astype(jnp.bfloat16)

    @pl.when(jnp.logical_and(i >= _NW, i < _NW + _NB))
    def _phase_a():
        k = i - _NW
        adjbf_ref[pl.ds(k * _BM, _BM), :] = adj_ref[...].astype(jnp.bfloat16)
        t = jnp.dot(adjbf_ref[pl.ds(k * _BM, _BM), :], xbf_ref[...],
                    preferred_element_type=jnp.float32)
        u = jnp.dot(t.astype(jnp.bfloat16), w1bf_ref[...],
                    preferred_element_type=jnp.float32) + b1_ref[...]
        h = jnp.maximum(u, 0.0)
        g = jnp.dot(h.astype(jnp.bfloat16), w2_ref[...],
                    preferred_element_type=jnp.float32)
        g_ref[pl.ds(k * _BM, _BM), :] = g.astype(jnp.bfloat16)

    @pl.when(i >= _NW + _NB)
    def _phase_b():
        k = i - _NW - _NB
        a = adjbf_ref[pl.ds(k * _BM2, _BM2), :]
        acc = jnp.dot(a, g_ref[...], preferred_element_type=jnp.float32)
        out_ref[...] = jnp.tanh(acc + b2_ref[...])


def kernel(x, adj, W1, b1, W2, b2):
    n, d_in = x.shape
    d_hid = W1.shape[1]
    bit = W2.shape[1]
    b1r = b1.reshape(1, d_hid)
    b2r = b2.reshape(1, bit)
    w2_b = W2.astype(jnp.bfloat16)

    out = pl.pallas_call(
        _body,
        grid=(_NW + _NB + _NB2,),
        in_specs=[
            pl.BlockSpec((_BM, n), lambda i: (jnp.clip(i - _NW, 0, _NB - 1), 0)),
            pl.BlockSpec((_XC, d_in), lambda i: (jnp.minimum(i, _NW - 1), 0)),
            pl.BlockSpec((_WC, d_hid), lambda i: (jnp.minimum(i, _NW - 1), 0)),
            pl.BlockSpec((1, d_hid), lambda i: (0, 0)),
            pl.BlockSpec((d_hid, bit), lambda i: (0, 0)),
            pl.BlockSpec((1, bit), lambda i: (0, 0)),
        ],
        out_specs=pl.BlockSpec((_BM2, bit),
                               lambda i: (jnp.clip(i - _NW - _NB, 0, _NB2 - 1), 0)),
        out_shape=jax.ShapeDtypeStruct((n, bit), jnp.float32),
        compiler_params=pltpu.CompilerParams(vmem_limit_bytes=100 * 1024 * 1024),
        scratch_shapes=[
            pltpu.VMEM((n, n), jnp.bfloat16),
            pltpu.VMEM((n, bit), jnp.bfloat16),
            pltpu.VMEM((n, d_in), jnp.bfloat16),
            pltpu.VMEM((d_in, d_hid), jnp.bfloat16),
        ],
    )(adj, x, W1, b1r, w2_b, b2r)
    return out
